# baseline (speedup 1.0000x reference)
import functools

import jax
import jax.numpy as jnp
from jax import lax
from jax.experimental import pallas as pl
from jax.experimental.pallas import tpu as pltpu
from jax.experimental.pallas import tpu_sc as plsc

NUM_EMB = 100000
D = 64
SEQ = 64
B = 4096
C = 11

NC = 2
NS = 16
NW = NC * NS
BROWS = B // NW
CHUNK = BROWS
NBUF = 4
KCH = SEQ * D // 128


def _gather_body(bh, table_hbm, idxt_hbm, out_hbm, idx_v, rows_v, s0, s1, s2, s3):
    sems = (s0, s1, s2, s3)
    brows = bh // NW
    wid = lax.axis_index("s") * NC + lax.axis_index("c")
    b0 = wid * brows
    pltpu.sync_copy(idxt_hbm.at[:, pl.ds(b0, brows)], idx_v)

    def _write(s, b):
        pltpu.sync_copy(
            rows_v.at[b],
            out_hbm.at[
                pl.ds((s // 2) * bh + b0, brows),
                pl.ds((s % 2) * D, D),
            ],
        )

    for b in range(NBUF):
        pltpu.async_copy(table_hbm.at[idx_v.at[b]], rows_v.at[b], sems[b])

    def body(i, _):
        for b in range(NBUF):
            s = i * NBUF + b
            pltpu.make_async_copy(
                table_hbm.at[idx_v.at[s]], rows_v.at[b], sems[b]
            ).wait()
            _write(s, b)
            pltpu.async_copy(
                table_hbm.at[idx_v.at[s + NBUF]], rows_v.at[b], sems[b]
            )
        return 0

    lax.fori_loop(0, SEQ // NBUF - 1, body, 0)

    for b in range(NBUF):
        s = SEQ - NBUF + b
        pltpu.make_async_copy(
            table_hbm.at[idx_v.at[s]], rows_v.at[b], sems[b]
        ).wait()
        _write(s, b)


@functools.lru_cache(maxsize=None)
def _make_gather(bh):
    return pl.kernel(
        functools.partial(_gather_body, bh),
        out_type=jax.ShapeDtypeStruct((KCH * bh, 128), jnp.float32),
        mesh=plsc.VectorSubcoreMesh(core_axis_name="c", subcore_axis_name="s"),
        scratch_types=[
            pltpu.VMEM((SEQ, bh // NW), jnp.int32),
            pltpu.VMEM((NBUF, bh // NW, D), jnp.float32),
            pltpu.SemaphoreType.DMA,
            pltpu.SemaphoreType.DMA,
            pltpu.SemaphoreType.DMA,
            pltpu.SemaphoreType.DMA,
        ],
        compiler_params=pltpu.CompilerParams(use_tc_tiling_on_sc=False),
    )


KSTEP = 4


def _mm_body(y_ref, w_ref, b_ref, o_ref):
    t = pl.program_id(0)
    bh = o_ref.shape[0]

    @pl.when(t == 0)
    def _():
        o_ref[...] = jnp.broadcast_to(b_ref[0:1, :], o_ref.shape)

    yb = jnp.concatenate(
        [y_ref[pl.ds(k * bh, bh), :] for k in range(KSTEP)], axis=1
    )
    o_ref[...] += jnp.dot(
        yb, w_ref[...], preferred_element_type=jnp.float32
    )


NSPLIT = 2
BH = B // NSPLIT


def _matmul(y, w_pad, b_pad, bh):
    return pl.pallas_call(
        _mm_body,
        grid=(KCH // KSTEP,),
        in_specs=[
            pl.BlockSpec((KSTEP * bh, 128), lambda t: (t, 0)),
            pl.BlockSpec((KSTEP * 128, 128), lambda t: (t, 0)),
            pl.BlockSpec((8, 128), lambda t: (0, 0)),
        ],
        out_specs=pl.BlockSpec((bh, 128), lambda t: (0, 0)),
        out_shape=jax.ShapeDtypeStruct((bh, 128), jnp.float32),
        compiler_params=pltpu.CompilerParams(
            dimension_semantics=("arbitrary",),
        ),
    )(y, w_pad, b_pad)


def kernel(input, table, fc_w, fc_b):
    idxt = input.astype(jnp.int32).T

    w_pad = (
        jnp.zeros((SEQ * D, 128), jnp.float32)
        .at[:, :C]
        .set(fc_w.T)
        .astype(jnp.bfloat16)
    )
    b_pad = jnp.zeros((8, 128), jnp.float32).at[:, :C].set(fc_b)

    g = _make_gather(BH)
    ys = [
        g(table, lax.slice(idxt, (0, h * BH), (SEQ, (h + 1) * BH)))
        for h in range(NSPLIT)
    ]
    outs = [_matmul(y, w_pad, b_pad, BH)[:, :C] for y in ys]
    return jnp.concatenate(outs, axis=0)


# speedup vs baseline: 1.0297x; 1.0297x over previous
import functools

import jax
import jax.numpy as jnp
from jax import lax
from jax.experimental import pallas as pl
from jax.experimental.pallas import tpu as pltpu
from jax.experimental.pallas import tpu_sc as plsc

NUM_EMB = 100000
D = 64
SEQ = 64
B = 4096
C = 11

NC = 2
NS = 16
NW = NC * NS
BROWS = B // NW
CHUNK = BROWS
NBUF = 8
KCH = SEQ * D // 128


def _gather_body(table_hbm, idxt_hbm, out_hbm, idx_v, rows_v,
                 s0, s1, s2, s3, s4, s5, s6, s7):
    sems = (s0, s1, s2, s3, s4, s5, s6, s7)
    wid = lax.axis_index("s") * NC + lax.axis_index("c")
    b0 = wid * BROWS
    pltpu.sync_copy(idxt_hbm.at[:, pl.ds(b0, BROWS)], idx_v)

    def _write(s, b):
        pltpu.sync_copy(
            rows_v.at[b],
            out_hbm.at[
                pl.ds((s // 2) * B + b0, BROWS),
                pl.ds((s % 2) * D, D),
            ],
        )

    for b in range(NBUF):
        pltpu.async_copy(table_hbm.at[idx_v.at[b]], rows_v.at[b], sems[b])

    def body(i, _):
        for b in range(NBUF):
            s = i * NBUF + b
            pltpu.make_async_copy(
                table_hbm.at[idx_v.at[s]], rows_v.at[b], sems[b]
            ).wait()
            _write(s, b)
            pltpu.async_copy(
                table_hbm.at[idx_v.at[s + NBUF]], rows_v.at[b], sems[b]
            )
        return 0

    lax.fori_loop(0, SEQ // NBUF - 1, body, 0)

    for b in range(NBUF):
        s = SEQ - NBUF + b
        pltpu.make_async_copy(
            table_hbm.at[idx_v.at[s]], rows_v.at[b], sems[b]
        ).wait()
        _write(s, b)


@functools.lru_cache(maxsize=None)
def _make_gather():
    return pl.kernel(
        _gather_body,
        out_type=jax.ShapeDtypeStruct((KCH * B, 128), jnp.float32),
        mesh=plsc.VectorSubcoreMesh(core_axis_name="c", subcore_axis_name="s"),
        scratch_types=[
            pltpu.VMEM((SEQ, BROWS), jnp.int32),
            pltpu.VMEM((NBUF, CHUNK, D), jnp.float32),
            pltpu.SemaphoreType.DMA,
            pltpu.SemaphoreType.DMA,
            pltpu.SemaphoreType.DMA,
            pltpu.SemaphoreType.DMA,
            pltpu.SemaphoreType.DMA,
            pltpu.SemaphoreType.DMA,
            pltpu.SemaphoreType.DMA,
            pltpu.SemaphoreType.DMA,
        ],
        compiler_params=pltpu.CompilerParams(use_tc_tiling_on_sc=False),
    )


KSTEP = 4


def _mm_body(y_ref, w_ref, b_ref, o_ref):
    t = pl.program_id(0)

    @pl.when(t == 0)
    def _():
        o_ref[...] = jnp.broadcast_to(b_ref[0:1, :], o_ref.shape)

    yb = jnp.concatenate(
        [y_ref[pl.ds(k * B, B), :] for k in range(KSTEP)], axis=1
    )
    o_ref[...] += jnp.dot(
        yb, w_ref[...], preferred_element_type=jnp.float32
    )


def kernel(input, table, fc_w, fc_b):
    idxt = input.astype(jnp.int32).T
    y = _make_gather()(table, idxt)

    w_pad = (
        jnp.zeros((SEQ * D, 128), jnp.float32)
        .at[:, :C]
        .set(fc_w.T)
        .astype(jnp.bfloat16)
    )
    b_pad = jnp.zeros((8, 128), jnp.float32).at[:, :C].set(fc_b)

    out_pad = pl.pallas_call(
        _mm_body,
        grid=(KCH // KSTEP,),
        in_specs=[
            pl.BlockSpec((KSTEP * B, 128), lambda t: (t, 0)),
            pl.BlockSpec((KSTEP * 128, 128), lambda t: (t, 0)),
            pl.BlockSpec((8, 128), lambda t: (0, 0)),
        ],
        out_specs=pl.BlockSpec((B, 128), lambda t: (0, 0)),
        out_shape=jax.ShapeDtypeStruct((B, 128), jnp.float32),
        compiler_params=pltpu.CompilerParams(
            dimension_semantics=("arbitrary",),
        ),
    )(y, w_pad, b_pad)
    return out_pad[:, :C]


# speedup vs baseline: 1.0402x; 1.0103x over previous
import functools

import jax
import jax.numpy as jnp
from jax import lax
from jax.experimental import pallas as pl
from jax.experimental.pallas import tpu as pltpu
from jax.experimental.pallas import tpu_sc as plsc

NUM_EMB = 100000
D = 64
SEQ = 64
B = 4096
C = 11

NC = 2
NS = 16
NW = NC * NS
BROWS = B // NW
CHUNK = BROWS
NBUF = 4
KCH = SEQ * D // 128


def _gather_body(table_hbm, idxt_hbm, out_hbm, idx_v, rows_v, s0, s1, s2, s3):
    sems = (s0, s1, s2, s3)
    wid = lax.axis_index("s") * NC + lax.axis_index("c")
    b0 = wid * BROWS
    pltpu.sync_copy(idxt_hbm.at[:, pl.ds(b0, BROWS)], idx_v)

    def _write(s, b):
        pltpu.sync_copy(
            rows_v.at[b],
            out_hbm.at[
                pl.ds((s // 2) * B + b0, BROWS),
                pl.ds((s % 2) * D, D),
            ],
        )

    for b in range(NBUF):
        pltpu.async_copy(table_hbm.at[idx_v.at[b]], rows_v.at[b], sems[b])

    def body(i, _):
        for b in range(NBUF):
            s = i * NBUF + b
            pltpu.make_async_copy(
                table_hbm.at[idx_v.at[s]], rows_v.at[b], sems[b]
            ).wait()
            _write(s, b)
            pltpu.async_copy(
                table_hbm.at[idx_v.at[s + NBUF]], rows_v.at[b], sems[b]
            )
        return 0

    lax.fori_loop(0, SEQ // NBUF - 1, body, 0)

    for b in range(NBUF):
        s = SEQ - NBUF + b
        pltpu.make_async_copy(
            table_hbm.at[idx_v.at[s]], rows_v.at[b], sems[b]
        ).wait()
        _write(s, b)


@functools.lru_cache(maxsize=None)
def _make_gather():
    return pl.kernel(
        _gather_body,
        out_type=jax.ShapeDtypeStruct((KCH * B, 128), jnp.float32),
        mesh=plsc.VectorSubcoreMesh(core_axis_name="c", subcore_axis_name="s"),
        scratch_types=[
            pltpu.VMEM((SEQ, BROWS), jnp.int32),
            pltpu.VMEM((NBUF, CHUNK, D), jnp.float32),
            pltpu.SemaphoreType.DMA,
            pltpu.SemaphoreType.DMA,
            pltpu.SemaphoreType.DMA,
            pltpu.SemaphoreType.DMA,
        ],
        compiler_params=pltpu.CompilerParams(use_tc_tiling_on_sc=False),
    )


KSTEP = 4


def _mm_body(y_ref, w_ref, b_ref, o_ref):
    t = pl.program_id(0)

    @pl.when(t == 0)
    def _():
        o_ref[...] = jnp.broadcast_to(b_ref[0:1, :], o_ref.shape)

    yb = jnp.concatenate(
        [y_ref[pl.ds(k * B, B), :] for k in range(KSTEP)], axis=1
    )
    o_ref[...] += jnp.dot(
        yb, w_ref[...], preferred_element_type=jnp.float32
    )


def kernel(input, table, fc_w, fc_b):
    idxt = input.astype(jnp.int32).T
    y = _make_gather()(table, idxt)

    w_pad = (
        jnp.zeros((SEQ * D, 128), jnp.float32)
        .at[:, :C]
        .set(fc_w.T)
        .astype(jnp.bfloat16)
    )
    b_pad = jnp.zeros((8, 128), jnp.float32).at[:, :C].set(fc_b)

    out_pad = pl.pallas_call(
        _mm_body,
        grid=(KCH // KSTEP,),
        in_specs=[
            pl.BlockSpec((KSTEP * B, 128), lambda t: (t, 0)),
            pl.BlockSpec((KSTEP * 128, 128), lambda t: (t, 0)),
            pl.BlockSpec((8, 128), lambda t: (0, 0)),
        ],
        out_specs=pl.BlockSpec((B, 128), lambda t: (0, 0)),
        out_shape=jax.ShapeDtypeStruct((B, 128), jnp.float32),
        compiler_params=pltpu.CompilerParams(
            dimension_semantics=("arbitrary",),
        ),
    )(y, w_pad, b_pad)
    return out_pad[:, :C]
